# Initial kernel scaffold; baseline (speedup 1.0000x reference)
import jax
import jax.numpy as jnp
from jax.experimental import pallas as pl


def kernel(x_feature, adj, W1, b1, W2, b2):
    raise NotImplementedError("write your pallas kernel here")


# speedup vs baseline: 22.8135x; 22.8135x over previous
import functools

import jax
import jax.numpy as jnp
from jax import lax
from jax.experimental import pallas as pl
from jax.experimental.pallas import tpu as pltpu
from jax.experimental.pallas import tpu_sc as plsc

NC = 2
NS = 16
NW = NC * NS
LANES = 16
EB = 128


def _sc_mesh():
    return plsc.VectorSubcoreMesh(core_axis_name="c", subcore_axis_name="s")


def _deg_body(nb, tpt, dst_ref, out_ref, idx_v, ones_v, zbuf_v, deg_sp):
    c = lax.axis_index("c")
    s = lax.axis_index("s")
    wid = c * NS + s
    for k in range(EB // LANES):
        ones_v[pl.ds(k * LANES, LANES)] = jnp.ones((LANES,), jnp.float32)
    for k in range(tpt // LANES):
        zbuf_v[pl.ds(k * LANES, LANES)] = jnp.zeros((LANES,), jnp.float32)
    pltpu.sync_copy(zbuf_v, deg_sp.at[pl.ds(s * tpt, tpt)])
    pltpu.sync_copy(dst_ref.at[wid], idx_v)
    plsc.subcore_barrier()

    def body(j, carry):
        pltpu.sync_copy(ones_v, deg_sp.at[idx_v.at[j]], add=True)
        return carry

    lax.fori_loop(0, nb, body, 0)
    plsc.subcore_barrier()
    pltpu.sync_copy(deg_sp.at[pl.ds(s * tpt, tpt)],
                    out_ref.at[c, pl.ds(s * tpt, tpt)])


def _make_deg(nb, np_):
    tpt = np_ // NS
    return pl.kernel(
        functools.partial(_deg_body, nb, tpt),
        out_type=jax.ShapeDtypeStruct((NC, np_), jnp.float32),
        mesh=_sc_mesh(),
        scratch_types=[
            pltpu.VMEM((nb, EB), jnp.int32),
            pltpu.VMEM((EB,), jnp.float32),
            pltpu.VMEM((tpt,), jnp.float32),
            pltpu.VMEM_SHARED((np_,), jnp.float32),
        ],
    )


def _prop_body(nb, tpt, f, stage, z_ref, src_ref, dst_ref, out_ref,
               sidx_v, didx_v, rows_v, sem, acc_sp, *maybe_z_sp):
    c = lax.axis_index("c")
    s = lax.axis_index("s")
    wid = c * NS + s

    def zrow(i, carry):
        for k in range(f // LANES):
            rows_v[i, pl.ds(k * LANES, LANES)] = jnp.zeros((LANES,), jnp.float32)
        return carry

    lax.fori_loop(0, EB, zrow, 0)
    for q in range(tpt // EB):
        pltpu.sync_copy(rows_v, acc_sp.at[pl.ds(s * tpt + q * EB, EB)])
    pltpu.sync_copy(src_ref.at[wid], sidx_v)
    pltpu.sync_copy(dst_ref.at[wid], didx_v)
    if stage:
        z_sp = maybe_z_sp[0]
        pltpu.sync_copy(z_ref.at[pl.ds(s * tpt, tpt)],
                        z_sp.at[pl.ds(s * tpt, tpt)])
        gather_src = z_sp
    else:
        gather_src = z_ref
    plsc.subcore_barrier()

    def body(j, carry):
        pltpu.async_copy(gather_src.at[sidx_v.at[j]], rows_v, sem).wait()
        pltpu.sync_copy(rows_v, acc_sp.at[didx_v.at[j]], add=True)
        return carry

    lax.fori_loop(0, nb, body, 0)
    plsc.subcore_barrier()
    pltpu.sync_copy(acc_sp.at[pl.ds(s * tpt, tpt)],
                    out_ref.at[c, pl.ds(s * tpt, tpt)])


def _make_prop(nb, np_, f):
    tpt = np_ // NS
    stage = f < 128
    scratch = [
        pltpu.VMEM((nb, EB), jnp.int32),
        pltpu.VMEM((nb, EB), jnp.int32),
        pltpu.VMEM((EB, f), jnp.float32),
        pltpu.SemaphoreType.DMA,
        pltpu.VMEM_SHARED((np_, f), jnp.float32),
    ]
    if stage:
        scratch.append(pltpu.VMEM_SHARED((np_, f), jnp.float32))
    return pl.kernel(
        functools.partial(_prop_body, nb, tpt, f, stage),
        out_type=jax.ShapeDtypeStruct((NC, np_, f), jnp.float32),
        mesh=_sc_mesh(),
        scratch_types=scratch,
    )


def _tc1_body(n, x_ref, w1_ref, dp_ref, z_ref, dinv_ref):
    deg = dp_ref[0, :] + dp_ref[1, :]
    dinv = jnp.where(deg > 0.0, lax.rsqrt(deg), 0.0)
    dinv_ref[...] = dinv
    y = jnp.dot(x_ref[...], w1_ref[...], preferred_element_type=jnp.float32)
    z_ref[:n, :] = y * dinv[:n, None]
    z_ref[n:, :] = jnp.zeros_like(z_ref[n:, :])


def _tc2_body(n, up_ref, dinv_ref, b1_ref, g_ref):
    dinv = dinv_ref[:n]
    u = up_ref[0, :n, :] + up_ref[1, :n, :]
    h = u * dinv[:, None] + b1_ref[...][None, :]
    h = jnp.maximum(h, 0.0)
    h = h - jnp.mean(h, axis=0, keepdims=True)
    h = h * lax.rsqrt(1e-6 + jnp.sum(h * h, axis=1, keepdims=True))
    g_ref[:n, :] = h * dinv[:, None]
    g_ref[n:, :] = jnp.zeros_like(g_ref[n:, :])


def _tc3_body(n, up_ref, dinv_ref, b2_ref, w2_ref, o_ref):
    dinv = dinv_ref[:n]
    u = up_ref[0, :n, :] + up_ref[1, :n, :]
    o = jnp.dot(u * dinv[:, None], w2_ref[...],
                preferred_element_type=jnp.float32) + b2_ref[...][None, :]
    o = o - jnp.mean(o, axis=0, keepdims=True)
    o = o * lax.rsqrt(1e-6 + jnp.sum(o * o, axis=1, keepdims=True))
    o_ref[...] = jax.nn.sigmoid(o)


def kernel(x_feature, adj, W1, b1, W2, b2):
    n, d = x_feature.shape
    h = W1.shape[1]
    c_out = W2.shape[1]
    e = adj.shape[1]
    np_ = n + 240
    assert np_ % (NS * 8) == 0
    fp2 = 16

    loop = jnp.arange(n, dtype=adj.dtype)
    src = jnp.concatenate([adj[0], loop])
    dst = jnp.concatenate([adj[1], loop])
    et = e + n
    nb = -(-et // (NW * EB))
    pad = NW * nb * EB - et
    padidx = n + (jnp.arange(pad, dtype=jnp.int32) % (np_ - n))
    srcp = jnp.concatenate([src, padidx]).reshape(NW, nb, EB)
    dstp = jnp.concatenate([dst, padidx]).reshape(NW, nb, EB)

    w2p = jnp.pad(W2, ((0, 0), (0, fp2 - c_out)))
    b2p = jnp.pad(b2, (0, fp2 - c_out))

    dp = _make_deg(nb, np_)(dstp)

    z1, dinv = pl.pallas_call(
        functools.partial(_tc1_body, n),
        out_shape=(jax.ShapeDtypeStruct((np_, h), jnp.float32),
                   jax.ShapeDtypeStruct((np_,), jnp.float32)),
    )(x_feature, W1, dp)

    u1 = _make_prop(nb, np_, h)(z1, srcp, dstp)

    g = pl.pallas_call(
        functools.partial(_tc2_body, n),
        out_shape=jax.ShapeDtypeStruct((np_, h), jnp.float32),
    )(u1, dinv, b1)

    u2 = _make_prop(nb, np_, h)(g, srcp, dstp)

    o = pl.pallas_call(
        functools.partial(_tc3_body, n),
        out_shape=jax.ShapeDtypeStruct((n, fp2), jnp.float32),
    )(u2, dinv, b2p, w2p)

    return o[:, :c_out]


# speedup vs baseline: 28.4128x; 1.2454x over previous
import functools

import jax
import jax.numpy as jnp
from jax import lax
from jax.experimental import pallas as pl
from jax.experimental.pallas import tpu as pltpu
from jax.experimental.pallas import tpu_sc as plsc

NC = 2
NS = 16
NW = NC * NS
LANES = 16
EB = 128


def _sc_mesh():
    return plsc.VectorSubcoreMesh(core_axis_name="c", subcore_axis_name="s")


def _deg_body(nb, tpt, dst_ref, out_ref, idx_v, ones_v, zbuf_v, deg_sp):
    c = lax.axis_index("c")
    s = lax.axis_index("s")
    wid = c * NS + s
    for k in range(EB // LANES):
        ones_v[pl.ds(k * LANES, LANES)] = jnp.ones((LANES,), jnp.float32)
    for k in range(tpt // LANES):
        zbuf_v[pl.ds(k * LANES, LANES)] = jnp.zeros((LANES,), jnp.float32)
    pltpu.sync_copy(zbuf_v, deg_sp.at[pl.ds(s * tpt, tpt)])
    pltpu.sync_copy(dst_ref.at[wid], idx_v)
    plsc.subcore_barrier()

    def body(j, carry):
        pltpu.sync_copy(ones_v, deg_sp.at[idx_v.at[j]], add=True)
        return carry

    lax.fori_loop(0, nb, body, 0)
    plsc.subcore_barrier()
    pltpu.sync_copy(deg_sp.at[pl.ds(s * tpt, tpt)],
                    out_ref.at[c, pl.ds(s * tpt, tpt)])


def _make_deg(nb, np_):
    tpt = np_ // NS
    return pl.kernel(
        functools.partial(_deg_body, nb, tpt),
        out_type=jax.ShapeDtypeStruct((NC, np_), jnp.float32),
        mesh=_sc_mesh(),
        scratch_types=[
            pltpu.VMEM((nb, EB), jnp.int32),
            pltpu.VMEM((EB,), jnp.float32),
            pltpu.VMEM((tpt,), jnp.float32),
            pltpu.VMEM_SHARED((np_,), jnp.float32),
        ],
    )


NBUF = 4


def _prop_body(nb, tpt, f, z_ref, src_ref, dst_ref, out_ref,
               sidx_v, didx_v, rows_v, gsem, acc_sp):
    c = lax.axis_index("c")
    s = lax.axis_index("s")
    wid = c * NS + s

    def zrow(i, carry):
        for k in range(f // LANES):
            rows_v[i, pl.ds(k * LANES, LANES)] = jnp.zeros((LANES,), jnp.float32)
        return carry

    lax.fori_loop(0, EB, zrow, 0)
    for q in range(tpt // EB):
        pltpu.sync_copy(rows_v, acc_sp.at[pl.ds(s * tpt + q * EB, EB)])
    pltpu.sync_copy(src_ref.at[wid], sidx_v)
    pltpu.sync_copy(dst_ref.at[wid], didx_v)
    plsc.subcore_barrier()

    def body(v, carry):
        pltpu.async_copy(z_ref.at[sidx_v.at[v]], rows_v, gsem).wait()
        pltpu.sync_copy(rows_v, acc_sp.at[didx_v.at[v]], add=True)
        return carry

    lax.fori_loop(0, nb, body, 0)
    plsc.subcore_barrier()
    pltpu.sync_copy(acc_sp.at[pl.ds(s * tpt, tpt)],
                    out_ref.at[c, pl.ds(s * tpt, tpt)])


def _make_prop(nb, np_, f, tc_tiling=True):
    tpt = np_ // NS
    return pl.kernel(
        functools.partial(_prop_body, nb, tpt, f),
        out_type=jax.ShapeDtypeStruct((NC, np_, f), jnp.float32),
        mesh=_sc_mesh(),
        scratch_types=[
            pltpu.VMEM((nb, EB), jnp.int32),
            pltpu.VMEM((nb, EB), jnp.int32),
            pltpu.VMEM((EB, f), jnp.float32),
            pltpu.SemaphoreType.DMA,
            pltpu.VMEM_SHARED((np_, f), jnp.float32),
        ],
        compiler_params=None if tc_tiling else pltpu.CompilerParams(
            use_tc_tiling_on_sc=False),
    )


def _tc1_body(n, x_ref, w1_ref, dp_ref, z_ref, dinv_ref):
    deg = dp_ref[0, :] + dp_ref[1, :]
    dinv = jnp.where(deg > 0.0, lax.rsqrt(deg), 0.0)
    dinv_ref[...] = dinv
    y = jnp.dot(x_ref[...], w1_ref[...], preferred_element_type=jnp.float32)
    z_ref[:n, :] = y * dinv[:n, None]
    z_ref[n:, :] = jnp.zeros_like(z_ref[n:, :])


def _tc2_body(n, up_ref, dinv_ref, b1_ref, w2_ref, z2_ref):
    dinv = dinv_ref[:n]
    u = up_ref[0, :n, :] + up_ref[1, :n, :]
    h = u * dinv[:, None] + b1_ref[...][None, :]
    h = jnp.maximum(h, 0.0)
    h = h - jnp.mean(h, axis=0, keepdims=True)
    h = h * lax.rsqrt(1e-6 + jnp.sum(h * h, axis=1, keepdims=True))
    z2 = jnp.dot(h, w2_ref[...], preferred_element_type=jnp.float32)
    z2_ref[:n, :] = z2 * dinv[:, None]
    z2_ref[n:, :] = jnp.zeros_like(z2_ref[n:, :])


def _tc3_body(n, up_ref, dinv_ref, b2_ref, o_ref):
    dinv = dinv_ref[:n]
    u = up_ref[0, :n, :] + up_ref[1, :n, :]
    o = u * dinv[:, None] + b2_ref[...][None, :]
    o = o - jnp.mean(o, axis=0, keepdims=True)
    o = o * lax.rsqrt(1e-6 + jnp.sum(o * o, axis=1, keepdims=True))
    o_ref[...] = jax.nn.sigmoid(o)


def kernel(x_feature, adj, W1, b1, W2, b2):
    n, d = x_feature.shape
    h = W1.shape[1]
    c_out = W2.shape[1]
    e = adj.shape[1]
    np_ = n + 240
    assert np_ % (NS * 8) == 0
    fp2 = 16

    loop = jnp.arange(n, dtype=adj.dtype)
    src = jnp.concatenate([adj[0], loop])
    dst = jnp.concatenate([adj[1], loop])
    et = e + n
    nb = -(-et // (NW * EB))
    pad = NW * nb * EB - et
    padidx = n + (jnp.arange(pad, dtype=jnp.int32) % (np_ - n))
    srcp = jnp.concatenate([src, padidx]).reshape(NW, nb, EB)
    dstp = jnp.concatenate([dst, padidx]).reshape(NW, nb, EB)

    w2p = jnp.pad(W2, ((0, 0), (0, fp2 - c_out)))
    b2p = jnp.pad(b2, (0, fp2 - c_out))

    dp = _make_deg(nb, np_)(dstp)

    z1, dinv = pl.pallas_call(
        functools.partial(_tc1_body, n),
        out_shape=(jax.ShapeDtypeStruct((np_, h), jnp.float32),
                   jax.ShapeDtypeStruct((np_,), jnp.float32)),
    )(x_feature, W1, dp)

    u1 = _make_prop(nb, np_, h)(z1, srcp, dstp)

    z2 = pl.pallas_call(
        functools.partial(_tc2_body, n),
        out_shape=jax.ShapeDtypeStruct((np_, fp2), jnp.float32),
    )(u1, dinv, b1, w2p)

    u2 = _make_prop(nb, np_, fp2, tc_tiling=False)(z2, srcp, dstp)

    o = pl.pallas_call(
        functools.partial(_tc3_body, n),
        out_shape=jax.ShapeDtypeStruct((n, fp2), jnp.float32),
    )(u2, dinv, b2p)

    return o[:, :c_out]


# speedup vs baseline: 29.2212x; 1.0285x over previous
import functools

import jax
import jax.numpy as jnp
from jax import lax
from jax.experimental import pallas as pl
from jax.experimental.pallas import tpu as pltpu
from jax.experimental.pallas import tpu_sc as plsc

NC = 2
NS = 16
NW = NC * NS
LANES = 16
EB = 128


def _sc_mesh():
    return plsc.VectorSubcoreMesh(core_axis_name="c", subcore_axis_name="s")


def _deg_body(nb, tpt, dst_ref, out_ref, idx_v, ones_v, zbuf_v, deg_sp):
    c = lax.axis_index("c")
    s = lax.axis_index("s")
    wid = c * NS + s
    for k in range(EB // LANES):
        ones_v[pl.ds(k * LANES, LANES)] = jnp.ones((LANES,), jnp.float32)
    for k in range(tpt // LANES):
        zbuf_v[pl.ds(k * LANES, LANES)] = jnp.zeros((LANES,), jnp.float32)
    pltpu.sync_copy(zbuf_v, deg_sp.at[pl.ds(s * tpt, tpt)])
    pltpu.sync_copy(dst_ref.at[wid], idx_v)
    plsc.subcore_barrier()

    def body(j, carry):
        pltpu.sync_copy(ones_v, deg_sp.at[idx_v.at[j]], add=True)
        return carry

    lax.fori_loop(0, nb, body, 0)
    plsc.subcore_barrier()
    pltpu.sync_copy(deg_sp.at[pl.ds(s * tpt, tpt)],
                    out_ref.at[c, pl.ds(s * tpt, tpt)])


def _make_deg(nb, np_):
    tpt = np_ // NS
    return pl.kernel(
        functools.partial(_deg_body, nb, tpt),
        out_type=jax.ShapeDtypeStruct((NC, np_), jnp.float32),
        mesh=_sc_mesh(),
        scratch_types=[
            pltpu.VMEM((nb, EB), jnp.int32),
            pltpu.VMEM((EB,), jnp.float32),
            pltpu.VMEM((tpt,), jnp.float32),
            pltpu.VMEM_SHARED((np_,), jnp.float32),
        ],
    )


NBUF = 4


def _prop_body(nb, tpt, f, skew, z_ref, src_ref, dst_ref, out_ref,
               sidx_v, didx_v, rows_v, gsem, acc_sp):
    c = lax.axis_index("c")
    s = lax.axis_index("s")
    wid = c * NS + s

    def zrow(i, carry):
        for k in range(f // LANES):
            rows_v[i, pl.ds(k * LANES, LANES)] = jnp.zeros((LANES,), jnp.float32)
        return carry

    lax.fori_loop(0, (2 * EB) if skew else EB, zrow, 0)
    if skew:
        for q in range(tpt // EB):
            pltpu.sync_copy(rows_v.at[pl.ds(0, EB)],
                            acc_sp.at[pl.ds(s * tpt + q * EB, EB)])
    else:
        for q in range(tpt // EB):
            pltpu.sync_copy(rows_v, acc_sp.at[pl.ds(s * tpt + q * EB, EB)])
    pltpu.sync_copy(src_ref.at[wid], sidx_v)
    pltpu.sync_copy(dst_ref.at[wid], didx_v)
    plsc.subcore_barrier()

    if skew:
        def body(v, carry):
            vv = jnp.minimum(v, nb - 1)
            wprev = jnp.maximum(v - 1, 0)
            gslot = (v % 2) * EB
            sslot = ((v + 1) % 2) * EB
            hdl = pltpu.async_copy(z_ref.at[sidx_v.at[vv]],
                                   rows_v.at[pl.ds(gslot, EB)], gsem)
            pltpu.sync_copy(rows_v.at[pl.ds(sslot, EB)],
                            acc_sp.at[didx_v.at[wprev]], add=True)
            hdl.wait()
            return carry

        lax.fori_loop(0, nb + 1, body, 0)
    else:
        def body(v, carry):
            pltpu.async_copy(z_ref.at[sidx_v.at[v]], rows_v, gsem).wait()
            pltpu.sync_copy(rows_v, acc_sp.at[didx_v.at[v]], add=True)
            return carry

        lax.fori_loop(0, nb, body, 0)
    plsc.subcore_barrier()
    pltpu.sync_copy(acc_sp.at[pl.ds(s * tpt, tpt)],
                    out_ref.at[c, pl.ds(s * tpt, tpt)])


def _make_prop(nb, np_, f, tc_tiling=True, skew=False):
    tpt = np_ // NS
    return pl.kernel(
        functools.partial(_prop_body, nb, tpt, f, skew),
        out_type=jax.ShapeDtypeStruct((NC, np_, f), jnp.float32),
        mesh=_sc_mesh(),
        scratch_types=[
            pltpu.VMEM((nb, EB), jnp.int32),
            pltpu.VMEM((nb, EB), jnp.int32),
            pltpu.VMEM(((2 * EB) if skew else EB, f), jnp.float32),
            pltpu.SemaphoreType.DMA,
            pltpu.VMEM_SHARED((np_, f), jnp.float32),
        ],
        compiler_params=None if tc_tiling else pltpu.CompilerParams(
            use_tc_tiling_on_sc=False),
    )


def _tc1_body(n, x_ref, w1_ref, dp_ref, z_ref, dinv_ref):
    deg = dp_ref[0, :] + dp_ref[1, :]
    dinv = jnp.where(deg > 0.0, lax.rsqrt(deg), 0.0)
    dinv_ref[...] = dinv
    y = jnp.dot(x_ref[...], w1_ref[...], preferred_element_type=jnp.float32)
    z_ref[:n, :] = y * dinv[:n, None]
    z_ref[n:, :] = jnp.zeros_like(z_ref[n:, :])


def _tc2_body(n, up_ref, dinv_ref, b1_ref, w2_ref, z2_ref):
    dinv = dinv_ref[:n]
    u = up_ref[0, :n, :] + up_ref[1, :n, :]
    h = u * dinv[:, None] + b1_ref[...][None, :]
    h = jnp.maximum(h, 0.0)
    h = h - jnp.mean(h, axis=0, keepdims=True)
    h = h * lax.rsqrt(1e-6 + jnp.sum(h * h, axis=1, keepdims=True))
    z2 = jnp.dot(h, w2_ref[...], preferred_element_type=jnp.float32)
    z2_ref[:n, :] = z2 * dinv[:, None]
    z2_ref[n:, :] = jnp.zeros_like(z2_ref[n:, :])


def _tc3_body(n, up_ref, dinv_ref, b2_ref, o_ref):
    dinv = dinv_ref[:n]
    u = up_ref[0, :n, :] + up_ref[1, :n, :]
    o = u * dinv[:, None] + b2_ref[...][None, :]
    o = o - jnp.mean(o, axis=0, keepdims=True)
    o = o * lax.rsqrt(1e-6 + jnp.sum(o * o, axis=1, keepdims=True))
    o_ref[...] = jax.nn.sigmoid(o)


def kernel(x_feature, adj, W1, b1, W2, b2):
    n, d = x_feature.shape
    h = W1.shape[1]
    c_out = W2.shape[1]
    e = adj.shape[1]
    np_ = n + 240
    assert np_ % (NS * 8) == 0
    fp2 = 16

    loop = jnp.arange(n, dtype=adj.dtype)
    src = jnp.concatenate([adj[0], loop])
    dst = jnp.concatenate([adj[1], loop])
    et = e + n
    nb = -(-et // (NW * EB))
    pad = NW * nb * EB - et
    padidx = n + (jnp.arange(pad, dtype=jnp.int32) % (np_ - n))
    srcp = jnp.concatenate([src, padidx]).reshape(NW, nb, EB)
    dstp = jnp.concatenate([dst, padidx]).reshape(NW, nb, EB)

    w2p = jnp.pad(W2, ((0, 0), (0, fp2 - c_out)))
    b2p = jnp.pad(b2, (0, fp2 - c_out))

    dp = _make_deg(nb, np_)(dstp)

    z1, dinv = pl.pallas_call(
        functools.partial(_tc1_body, n),
        out_shape=(jax.ShapeDtypeStruct((np_, h), jnp.float32),
                   jax.ShapeDtypeStruct((np_,), jnp.float32)),
    )(x_feature, W1, dp)

    u1 = _make_prop(nb, np_, h)(z1, srcp, dstp)

    z2 = pl.pallas_call(
        functools.partial(_tc2_body, n),
        out_shape=jax.ShapeDtypeStruct((np_, fp2), jnp.float32),
    )(u1, dinv, b1, w2p)

    u2 = _make_prop(nb, np_, fp2, tc_tiling=False, skew=True)(z2, srcp, dstp)

    o = pl.pallas_call(
        functools.partial(_tc3_body, n),
        out_shape=jax.ShapeDtypeStruct((n, fp2), jnp.float32),
    )(u2, dinv, b2p)

    return o[:, :c_out]
